# Initial kernel scaffold; baseline (speedup 1.0000x reference)
import jax
import jax.numpy as jnp
from jax.experimental import pallas as pl


def kernel(x_real, edge_index_real, batch_real, edge_attr_real, x_gen, edge_index_gen, batch_gen, edge_attr_gen, W_node, b_node, W_edge, b_edge, W1, b1, W2, b2):
    raise NotImplementedError("write your pallas kernel here")


# speedup vs baseline: 1.8320x; 1.8320x over previous
import functools

import jax
import jax.numpy as jnp
from jax import lax
from jax.experimental import pallas as pl
from jax.experimental.pallas import tpu as pltpu
from jax.experimental.pallas import tpu_sc as plsc

N = 10000
E = 320000
DN = 128
DE = 16
NG = 64
HID = 35
HP = 48
NN = 2 * N
ROWS = 2 * E
NW = 32
PER_TILE = ROWS // NW
CHUNK = 128
NFULL = PER_TILE // CHUNK
TAIL = PER_TILE - NFULL * CHUNK
SUB_ROWS = NN // 16

def _sc_mesh():
    return plsc.VectorSubcoreMesh(core_axis_name="c", subcore_axis_name="s",
                                  num_cores=2, num_subcores=16)


@functools.cache
def _sc_gather_kernel():
    return functools.partial(
        pl.kernel,
        out_type=jax.ShapeDtypeStruct((ROWS, HP), jnp.float32),
        mesh=_sc_mesh(),
        compiler_params=pltpu.CompilerParams(use_tc_tiling_on_sc=False),
        scratch_types=[
            pltpu.VMEM((CHUNK,), jnp.int32),
            pltpu.VMEM((CHUNK, HP), jnp.float32),
            pltpu.VMEM((TAIL,), jnp.int32),
            pltpu.VMEM((TAIL, HP), jnp.float32),
            pltpu.SemaphoreType.DMA,
        ],
    )(_sc_gather_body)


def _sc_gather(table, src):
    return _sc_gather_kernel()(table, src)


def _sc_gather_body(table_hbm, src_hbm, out_hbm, idx_v, rows_v, idx_t, rows_t, sem):
    wid = lax.axis_index("s") * 2 + lax.axis_index("c")
    base0 = wid * PER_TILE

    def body(i, _):
        base = base0 + i * CHUNK
        pltpu.sync_copy(src_hbm.at[pl.ds(base, CHUNK)], idx_v)
        pltpu.async_copy(table_hbm.at[idx_v], rows_v, sem).wait()
        pltpu.sync_copy(rows_v, out_hbm.at[pl.ds(base, CHUNK)])
        return 0

    lax.fori_loop(0, NFULL, body, 0)
    base = base0 + NFULL * CHUNK
    pltpu.sync_copy(src_hbm.at[pl.ds(base, TAIL)], idx_t)
    pltpu.async_copy(table_hbm.at[idx_t], rows_t, sem).wait()
    pltpu.sync_copy(rows_t, out_hbm.at[pl.ds(base, TAIL)])


@functools.cache
def _sc_scatter_kernel():
    return functools.partial(
        pl.kernel,
        out_type=jax.ShapeDtypeStruct((2, NN, HP), jnp.float32),
        mesh=_sc_mesh(),
        compiler_params=pltpu.CompilerParams(use_tc_tiling_on_sc=False),
        scratch_types=[
            pltpu.VMEM((CHUNK,), jnp.int32),
            pltpu.VMEM((CHUNK, HP), jnp.float32),
            pltpu.VMEM((TAIL,), jnp.int32),
            pltpu.VMEM((TAIL, HP), jnp.float32),
            pltpu.VMEM((SUB_ROWS, HP), jnp.float32),
            pltpu.VMEM_SHARED((NN, HP), jnp.float32),
            pltpu.SemaphoreType.DMA,
        ],
    )(_sc_scatter_body)


def _sc_scatter(msg, dst):
    return _sc_scatter_kernel()(msg, dst)


def _sc_scatter_body(msg_hbm, dst_hbm, out_hbm, idx_v, rows_v, idx_t, rows_t, zbuf, acc, sem):
    cid = lax.axis_index("c")
    sid = lax.axis_index("s")
    wid = sid * 2 + cid
    base0 = wid * PER_TILE

    def zbody(r, _):
        zrow = jnp.zeros((16,), jnp.float32)
        zbuf[r, pl.ds(0, 16)] = zrow
        zbuf[r, pl.ds(16, 16)] = zrow
        zbuf[r, pl.ds(32, 16)] = zrow
        return 0

    lax.fori_loop(0, SUB_ROWS, zbody, 0)
    pltpu.sync_copy(zbuf, acc.at[pl.ds(sid * SUB_ROWS, SUB_ROWS)])
    plsc.subcore_barrier()

    def body(i, _):
        base = base0 + i * CHUNK
        pltpu.sync_copy(dst_hbm.at[pl.ds(base, CHUNK)], idx_v)
        pltpu.sync_copy(msg_hbm.at[pl.ds(base, CHUNK)], rows_v)
        pltpu.sync_copy(rows_v, acc.at[idx_v], add=True)
        return 0

    lax.fori_loop(0, NFULL, body, 0)
    base = base0 + NFULL * CHUNK
    pltpu.sync_copy(dst_hbm.at[pl.ds(base, TAIL)], idx_t)
    pltpu.sync_copy(msg_hbm.at[pl.ds(base, TAIL)], rows_t)
    pltpu.sync_copy(rows_t, acc.at[idx_t], add=True)

    plsc.subcore_barrier()
    pltpu.sync_copy(acc.at[pl.ds(sid * SUB_ROWS, SUB_ROWS)],
                    out_hbm.at[cid, pl.ds(sid * SUB_ROWS, SUB_ROWS)])


def _enc_node_body(x_ref, w_ref, b_ref, o_ref):
    o_ref[...] = jnp.dot(x_ref[...], w_ref[...],
                         preferred_element_type=jnp.float32) + b_ref[...]


def _enc_node(x, w, b):
    bn = 2000
    return pl.pallas_call(
        _enc_node_body,
        grid=(NN // bn,),
        in_specs=[
            pl.BlockSpec((bn, DN), lambda i: (i, 0)),
            pl.BlockSpec((DN, HP), lambda i: (0, 0)),
            pl.BlockSpec((1, HP), lambda i: (0, 0)),
        ],
        out_specs=pl.BlockSpec((bn, HP), lambda i: (i, 0)),
        out_shape=jax.ShapeDtypeStruct((NN, HP), jnp.float32),
    )(x, w, b)


def _msg_body(g_ref, ea_ref, w_ref, b_ref, o_ref):
    ea = jnp.dot(ea_ref[...], w_ref[...],
                 preferred_element_type=jnp.float32) + b_ref[...]
    o_ref[...] = jnp.maximum(g_ref[...] + ea, 0.0)


def _msg(gat, edge_attr, w, b):
    be = 4000
    return pl.pallas_call(
        _msg_body,
        grid=(ROWS // be,),
        in_specs=[
            pl.BlockSpec((be, HP), lambda i: (i, 0)),
            pl.BlockSpec((be, DE), lambda i: (i, 0)),
            pl.BlockSpec((DE, HP), lambda i: (0, 0)),
            pl.BlockSpec((1, HP), lambda i: (0, 0)),
        ],
        out_specs=pl.BlockSpec((be, HP), lambda i: (i, 0)),
        out_shape=jax.ShapeDtypeStruct((ROWS, HP), jnp.float32),
    )(gat, edge_attr, w, b)


def _mlp_body(h_ref, p0_ref, p1_ref, w1_ref, b1_ref, w2_ref, b2_ref, o_ref):
    t = h_ref[...] + p0_ref[...] + p1_ref[...]
    r = jnp.maximum(jnp.dot(t, w1_ref[...],
                            preferred_element_type=jnp.float32) + b1_ref[...], 0.0)
    o_ref[...] = jnp.dot(r, w2_ref[...],
                         preferred_element_type=jnp.float32) + b2_ref[...]


def _mlp(h, p0, p1, w1, b1, w2, b2):
    bn = 2000
    return pl.pallas_call(
        _mlp_body,
        grid=(NN // bn,),
        in_specs=[
            pl.BlockSpec((bn, HP), lambda i: (i, 0)),
            pl.BlockSpec((bn, HP), lambda i: (i, 0)),
            pl.BlockSpec((bn, HP), lambda i: (i, 0)),
            pl.BlockSpec((HP, 64), lambda i: (0, 0)),
            pl.BlockSpec((1, 64), lambda i: (0, 0)),
            pl.BlockSpec((64, HP), lambda i: (0, 0)),
            pl.BlockSpec((1, HP), lambda i: (0, 0)),
        ],
        out_specs=pl.BlockSpec((bn, HP), lambda i: (i, 0)),
        out_shape=jax.ShapeDtypeStruct((NN, HP), jnp.float32),
    )(h, p0, p1, w1, b1, w2, b2)


def _embed_body(b_ref, n1_ref, n2_ref, o_ref):
    i = pl.program_id(1)

    @pl.when(i == 0)
    def _():
        o_ref[...] = jnp.zeros_like(o_ref)

    iota = lax.broadcasted_iota(jnp.int32, (1, NG), 1).astype(jnp.float32)
    oh = (b_ref[...] == iota).astype(jnp.float32)
    dims = (((0,), (0,)), ((), ()))
    g1 = lax.dot_general(oh, n1_ref[...], dims,
                         preferred_element_type=jnp.float32)
    g2 = lax.dot_general(oh, n2_ref[...], dims,
                         preferred_element_type=jnp.float32)
    o_ref[0, :, 0:HP] += g1
    o_ref[0, :, HP:2 * HP] += g2


def _embed(batchf, nf1, nf2):
    bn = 2000
    nblk = N // bn
    return pl.pallas_call(
        _embed_body,
        grid=(2, nblk),
        in_specs=[
            pl.BlockSpec((bn, 1), lambda g, i: (g * nblk + i, 0)),
            pl.BlockSpec((bn, HP), lambda g, i: (g * nblk + i, 0)),
            pl.BlockSpec((bn, HP), lambda g, i: (g * nblk + i, 0)),
        ],
        out_specs=pl.BlockSpec((1, NG, 2 * HP), lambda g, i: (g, 0, 0)),
        out_shape=jax.ShapeDtypeStruct((2, NG, 2 * HP), jnp.float32),
    )(batchf, nf1, nf2)


def _mmd_body(emb_ref, o_ref):
    def term(xi, yi):
        X = emb_ref[xi]

        def body(j, acc):
            row = emb_ref[yi, pl.ds(j, 1), :]
            diff = X - row
            s = jnp.sum(diff * diff, axis=1, keepdims=True)
            safe = jnp.where(s > 0, s, 1.0)
            d = jnp.where(s > 0, jnp.sqrt(safe), 0.0)
            return acc + jnp.sum(jnp.exp(-0.5 * d))
        return lax.fori_loop(0, NG, body, jnp.float32(0.0))

    den = float(2 * HID) * float(2 * HID)
    total = (term(0, 0) + term(1, 1) - 2.0 * term(0, 1)) / den
    o_ref[...] = jnp.broadcast_to(total, (1, 1))


def _mmd(emb):
    return pl.pallas_call(
        _mmd_body,
        out_shape=jax.ShapeDtypeStruct((1, 1), jnp.float32),
    )(emb)


def kernel(x_real, edge_index_real, batch_real, edge_attr_real,
           x_gen, edge_index_gen, batch_gen, edge_attr_gen,
           W_node, b_node, W_edge, b_edge, W1, b1, W2, b2):
    pad = HP - HID
    wn = jnp.pad(W_node, ((0, 0), (0, pad)))
    bn = jnp.pad(b_node, (0, pad)).reshape(1, HP)
    we = jnp.pad(W_edge, ((0, 0), (0, pad)))
    be = jnp.pad(b_edge, (0, pad)).reshape(1, HP)
    w1 = jnp.pad(W1, ((0, pad), (0, 0)))
    b1r = b1.reshape(1, 64)
    w2 = jnp.pad(W2, ((0, 0), (0, pad)))
    b2r = jnp.pad(b2, (0, pad)).reshape(1, HP)

    x_all = jnp.concatenate([x_real, x_gen], axis=0)
    ea_all = jnp.concatenate([edge_attr_real, edge_attr_gen], axis=0)
    src_all = jnp.concatenate([edge_index_real[0], edge_index_gen[0] + N])
    dst_all = jnp.concatenate([edge_index_real[1], edge_index_gen[1] + N])
    batchf = jnp.concatenate([batch_real, batch_gen]).astype(jnp.float32)
    batchf = batchf.reshape(NN, 1)

    h = _enc_node(x_all, wn, bn)

    gat1 = _sc_gather(h, src_all)
    msg1 = _msg(gat1, ea_all, we, be)
    parts1 = _sc_scatter(msg1, dst_all)
    nf1 = _mlp(h, parts1[0], parts1[1], w1, b1r, w2, b2r)

    gat2 = _sc_gather(nf1, src_all)
    msg2 = _msg(gat2, ea_all, we, be)
    parts2 = _sc_scatter(msg2, dst_all)
    nf2 = _mlp(nf1, parts2[0], parts2[1], w1, b1r, w2, b2r)

    emb = _embed(batchf, nf1, nf2)
    return _mmd(emb)[0, 0]


# speedup vs baseline: 2.7099x; 1.4792x over previous
import functools

import jax
import jax.numpy as jnp
from jax import lax
from jax.experimental import pallas as pl
from jax.experimental.pallas import tpu as pltpu
from jax.experimental.pallas import tpu_sc as plsc

N = 10000
E = 320000
DN = 128
DE = 16
NG = 64
HID = 35
HP = 48
NN = 2 * N
ROWS = 2 * E
NW = 32
CH = 128
K = 2
GROUP = K * CH
CPT = 160
NGRP = CPT // K
EPG_P = 16 * CPT * CH
CPG = EPG_P // CH
ROWS_P = 2 * EPG_P
NCHUNK = ROWS_P // CH
TRASH = N
ACC_ROWS = N + 16
SUB_ROWS = ACC_ROWS // 16


def _sc_mesh():
    return plsc.VectorSubcoreMesh(core_axis_name="c", subcore_axis_name="s",
                                  num_cores=2, num_subcores=16)


@functools.cache
def _sc_layer_kernel():
    return functools.partial(
        pl.kernel,
        out_type=jax.ShapeDtypeStruct((2, ACC_ROWS, HP), jnp.float32),
        mesh=_sc_mesh(),
        compiler_params=pltpu.CompilerParams(use_tc_tiling_on_sc=False),
        scratch_types=[
            pltpu.VMEM((2, K, CH), jnp.int32),
            pltpu.VMEM((2, K, CH), jnp.int32),
            pltpu.VMEM((2, GROUP, HP), jnp.float32),
            pltpu.VMEM((2, GROUP, HP), jnp.float32),
            pltpu.VMEM_SHARED((ACC_ROWS, HP), jnp.float32),
            pltpu.SemaphoreType.DMA,
            pltpu.SemaphoreType.DMA,
            pltpu.SemaphoreType.DMA,
        ],
    )(_sc_layer_body)


def _sc_layer(table, ea, src2d, dst2d):
    return _sc_layer_kernel()(table, ea, src2d, dst2d)


def _sc_layer_body(table_hbm, ea_hbm, src_hbm, dst_hbm, out_hbm,
                   idxs, idxd, gbuf, ebuf, acc, isem, gsem, esem):
    cid = lax.axis_index("c")
    sid = lax.axis_index("s")
    cb0 = cid * CPG + sid * CPT

    def zbody(r, _):
        zrow = jnp.zeros((16,), jnp.float32)
        gbuf[0, r, pl.ds(0, 16)] = zrow
        gbuf[0, r, pl.ds(16, 16)] = zrow
        gbuf[0, r, pl.ds(32, 16)] = zrow
        return 0

    lax.fori_loop(0, GROUP, zbody, 0)
    base = sid * SUB_ROWS
    pltpu.sync_copy(gbuf.at[0], acc.at[pl.ds(base, GROUP)])
    pltpu.sync_copy(gbuf.at[0], acc.at[pl.ds(base + GROUP, GROUP)])
    pltpu.sync_copy(gbuf.at[0].at[pl.ds(0, SUB_ROWS - 2 * GROUP)],
                    acc.at[pl.ds(base + 2 * GROUP, SUB_ROWS - 2 * GROUP)])
    plsc.subcore_barrier()

    def fire_idx(g, b):
        cb = cb0 + g * K
        pltpu.async_copy(src_hbm.at[pl.ds(cb, K)], idxs.at[b], isem)
        pltpu.async_copy(dst_hbm.at[pl.ds(cb, K)], idxd.at[b], isem)

    def drain_idx(g, b):
        cb = cb0 + g * K
        pltpu.make_async_copy(src_hbm.at[pl.ds(cb, K)], idxs.at[b], isem).wait()
        pltpu.make_async_copy(dst_hbm.at[pl.ds(cb, K)], idxd.at[b], isem).wait()

    def fire_gathers(b):
        for j in range(K):
            pltpu.async_copy(table_hbm.at[idxs.at[b].at[j]],
                             gbuf.at[b].at[pl.ds(j * CH, CH)], gsem)

    def drain_gathers(b):
        for j in range(K):
            pltpu.make_async_copy(table_hbm.at[idxs.at[b].at[j]],
                                  gbuf.at[b].at[pl.ds(j * CH, CH)], gsem).wait()

    def fire_ea(g, b):
        rb = (cb0 + g * K) * CH
        pltpu.async_copy(ea_hbm.at[pl.ds(rb, GROUP)], ebuf.at[b], esem)

    def drain_ea(g, b):
        rb = (cb0 + g * K) * CH
        pltpu.make_async_copy(ea_hbm.at[pl.ds(rb, GROUP)], ebuf.at[b], esem).wait()

    def compute(b):
        gb = gbuf.at[b]
        eb = ebuf.at[b]

        def crow(r, _):
            for c in range(3):
                sl = pl.ds(c * 16, 16)
                gb[r, sl] = jnp.maximum(gb[r, sl] + eb[r, sl], 0.0)
            return 0

        lax.fori_loop(0, GROUP, crow, 0, unroll=4)

    def scatters(b):
        for j in range(K):
            pltpu.sync_copy(gbuf.at[b].at[pl.ds(j * CH, CH)],
                            acc.at[idxd.at[b].at[j]], add=True)

    def slot(g, b, fire_next=True):
        nb = 1 - b
        if fire_next:
            fire_idx(g + 1, nb)
        drain_gathers(b)
        drain_ea(g, b)
        if fire_next:
            drain_idx(g + 1, nb)
            fire_gathers(nb)
            fire_ea(g + 1, nb)
        compute(b)
        scatters(b)

    fire_idx(0, 0)
    drain_idx(0, 0)
    fire_gathers(0)
    fire_ea(0, 0)

    slot(0, 0)
    slot(1, 1)

    def pair(p, _):
        g = 2 + 2 * p
        slot(g, 0)
        slot(g + 1, 1)
        return 0

    lax.fori_loop(0, (NGRP - 4) // 2, pair, 0)
    slot(NGRP - 2, 0)
    slot(NGRP - 1, 1, fire_next=False)

    plsc.subcore_barrier()
    pltpu.sync_copy(acc.at[pl.ds(sid * SUB_ROWS, SUB_ROWS)],
                    out_hbm.at[cid, pl.ds(sid * SUB_ROWS, SUB_ROWS)])


def _enc_node_body(x_ref, w_ref, b_ref, o_ref):
    o_ref[...] = jnp.dot(x_ref[...], w_ref[...],
                         preferred_element_type=jnp.float32) + b_ref[...]


def _enc_node(x, w, b):
    bn = 2000
    return pl.pallas_call(
        _enc_node_body,
        grid=(NN // bn,),
        in_specs=[
            pl.BlockSpec((bn, DN), lambda i: (i, 0)),
            pl.BlockSpec((DN, HP), lambda i: (0, 0)),
            pl.BlockSpec((1, HP), lambda i: (0, 0)),
        ],
        out_specs=pl.BlockSpec((bn, HP), lambda i: (i, 0)),
        out_shape=jax.ShapeDtypeStruct((NN, HP), jnp.float32),
    )(x, w, b)


def _enc_edge_body(ea_ref, w_ref, b_ref, o_ref):
    o_ref[...] = jnp.dot(ea_ref[...], w_ref[...],
                         preferred_element_type=jnp.float32) + b_ref[...]


def _enc_edge(ea, w, b):
    be = 2560
    nvb = E // be
    npb = EPG_P // be

    def in_map(i):
        return ((i // npb) * nvb + jnp.minimum(i % npb, nvb - 1), 0)

    return pl.pallas_call(
        _enc_edge_body,
        grid=(2 * npb,),
        in_specs=[
            pl.BlockSpec((be, DE), in_map),
            pl.BlockSpec((DE, HP), lambda i: (0, 0)),
            pl.BlockSpec((1, HP), lambda i: (0, 0)),
        ],
        out_specs=pl.BlockSpec((be, HP), lambda i: (i, 0)),
        out_shape=jax.ShapeDtypeStruct((ROWS_P, HP), jnp.float32),
    )(ea, w, b)


def _mlp_body(h_ref, p_ref, w1_ref, b1_ref, w2_ref, b2_ref, o_ref):
    t = h_ref[...] + p_ref[...]
    r = jnp.maximum(jnp.dot(t, w1_ref[...],
                            preferred_element_type=jnp.float32) + b1_ref[...], 0.0)
    o_ref[...] = jnp.dot(r, w2_ref[...],
                         preferred_element_type=jnp.float32) + b2_ref[...]


def _mlp(h, p, w1, b1, w2, b2):
    bn = 2000
    return pl.pallas_call(
        _mlp_body,
        grid=(NN // bn,),
        in_specs=[
            pl.BlockSpec((bn, HP), lambda i: (i, 0)),
            pl.BlockSpec((bn, HP), lambda i: (i, 0)),
            pl.BlockSpec((HP, 64), lambda i: (0, 0)),
            pl.BlockSpec((1, 64), lambda i: (0, 0)),
            pl.BlockSpec((64, HP), lambda i: (0, 0)),
            pl.BlockSpec((1, HP), lambda i: (0, 0)),
        ],
        out_specs=pl.BlockSpec((bn, HP), lambda i: (i, 0)),
        out_shape=jax.ShapeDtypeStruct((NN, HP), jnp.float32),
    )(h, p, w1, b1, w2, b2)


def _embed_body(b_ref, n1_ref, n2_ref, o_ref):
    i = pl.program_id(1)

    @pl.when(i == 0)
    def _():
        o_ref[...] = jnp.zeros_like(o_ref)

    iota = lax.broadcasted_iota(jnp.int32, (1, NG), 1).astype(jnp.float32)
    oh = (b_ref[...] == iota).astype(jnp.float32)
    dims = (((0,), (0,)), ((), ()))
    g1 = lax.dot_general(oh, n1_ref[...], dims,
                         preferred_element_type=jnp.float32)
    g2 = lax.dot_general(oh, n2_ref[...], dims,
                         preferred_element_type=jnp.float32)
    o_ref[0, :, 0:HP] += g1
    o_ref[0, :, HP:2 * HP] += g2


def _embed(batchf, nf1, nf2):
    bn = 2000
    nblk = N // bn
    return pl.pallas_call(
        _embed_body,
        grid=(2, nblk),
        in_specs=[
            pl.BlockSpec((bn, 1), lambda g, i: (g * nblk + i, 0)),
            pl.BlockSpec((bn, HP), lambda g, i: (g * nblk + i, 0)),
            pl.BlockSpec((bn, HP), lambda g, i: (g * nblk + i, 0)),
        ],
        out_specs=pl.BlockSpec((1, NG, 2 * HP), lambda g, i: (g, 0, 0)),
        out_shape=jax.ShapeDtypeStruct((2, NG, 2 * HP), jnp.float32),
    )(batchf, nf1, nf2)


def _mmd_body(emb_ref, o_ref):
    def term(xi, yi):
        X = emb_ref[xi]

        def body(j, acc):
            row = emb_ref[yi, pl.ds(j, 1), :]
            diff = X - row
            s = jnp.sum(diff * diff, axis=1, keepdims=True)
            safe = jnp.where(s > 0, s, 1.0)
            d = jnp.where(s > 0, jnp.sqrt(safe), 0.0)
            return acc + jnp.sum(jnp.exp(-0.5 * d))
        return lax.fori_loop(0, NG, body, jnp.float32(0.0))

    den = float(2 * HID) * float(2 * HID)
    total = (term(0, 0) + term(1, 1) - 2.0 * term(0, 1)) / den
    o_ref[...] = jnp.broadcast_to(total, (1, 1))


def _mmd(emb):
    return pl.pallas_call(
        _mmd_body,
        out_shape=jax.ShapeDtypeStruct((1, 1), jnp.float32),
    )(emb)


def kernel(x_real, edge_index_real, batch_real, edge_attr_real,
           x_gen, edge_index_gen, batch_gen, edge_attr_gen,
           W_node, b_node, W_edge, b_edge, W1, b1, W2, b2):
    pad = HP - HID
    wn = jnp.pad(W_node, ((0, 0), (0, pad)))
    bn = jnp.pad(b_node, (0, pad)).reshape(1, HP)
    we = jnp.pad(W_edge, ((0, 0), (0, pad)))
    be = jnp.pad(b_edge, (0, pad)).reshape(1, HP)
    w1 = jnp.pad(W1, ((0, pad), (0, 0)))
    b1r = b1.reshape(1, 64)
    w2 = jnp.pad(W2, ((0, 0), (0, pad)))
    b2r = jnp.pad(b2, (0, pad)).reshape(1, HP)

    x_all = jnp.concatenate([x_real, x_gen], axis=0)
    ea_all = jnp.concatenate([edge_attr_real, edge_attr_gen], axis=0)
    padn = EPG_P - E
    src2d = jnp.concatenate([
        jnp.pad(edge_index_real[0], (0, padn)),
        jnp.pad(edge_index_gen[0] + N, (0, padn)),
    ]).reshape(NCHUNK, CH)
    dst2d = jnp.concatenate([
        jnp.pad(edge_index_real[1], (0, padn), constant_values=TRASH),
        jnp.pad(edge_index_gen[1], (0, padn), constant_values=TRASH),
    ]).reshape(NCHUNK, CH)
    batchf = jnp.concatenate([batch_real, batch_gen]).astype(jnp.float32)
    batchf = batchf.reshape(NN, 1)

    h = _enc_node(x_all, wn, bn)
    eaf = _enc_edge(ea_all, we, be)

    def gine_layer(t, _):
        parts = _sc_layer(t, eaf, src2d, dst2d)
        p = parts[:, :N].reshape(NN, HP)
        nf = _mlp(t, p, w1, b1r, w2, b2r)
        return nf, nf

    _, nfs = lax.scan(gine_layer, h, None, length=2)
    nf1, nf2 = nfs[0], nfs[1]

    emb = _embed(batchf, nf1, nf2)
    return _mmd(emb)[0, 0]


# speedup vs baseline: 3.6228x; 1.3369x over previous
import functools

import jax
import jax.numpy as jnp
from jax import lax
from jax.experimental import pallas as pl
from jax.experimental.pallas import tpu as pltpu
from jax.experimental.pallas import tpu_sc as plsc

N = 10000
E = 320000
DN = 128
DE = 16
NG = 64
HID = 35
HP = 48
NN = 2 * N
ROWS = 2 * E
NW = 32
CH = 128
K = 2
GROUP = K * CH
CPT = 160
NGRP = CPT // K
EPG_P = 16 * CPT * CH
CPG = EPG_P // CH
ROWS_P = 2 * EPG_P
NCHUNK = ROWS_P // CH
TRASH = N
ACC_ROWS = N + 16
SUB_ROWS = ACC_ROWS // 16
TBL_SUB = N // 16


def _sc_mesh():
    return plsc.VectorSubcoreMesh(core_axis_name="c", subcore_axis_name="s",
                                  num_cores=2, num_subcores=16)


@functools.cache
def _sc_layer_kernel():
    return functools.partial(
        pl.kernel,
        out_type=jax.ShapeDtypeStruct((2, ACC_ROWS, HP), jnp.float32),
        mesh=_sc_mesh(),
        compiler_params=pltpu.CompilerParams(use_tc_tiling_on_sc=False),
        scratch_types=[
            pltpu.VMEM((2, K, CH), jnp.int32),
            pltpu.VMEM((2, K, CH), jnp.int32),
            pltpu.VMEM((2, GROUP, HP), jnp.float32),
            pltpu.VMEM((2, GROUP, HP), jnp.float32),
            pltpu.VMEM_SHARED((ACC_ROWS, HP), jnp.float32),
            pltpu.VMEM_SHARED((N, HP), jnp.float32),
            pltpu.SemaphoreType.DMA,
            pltpu.SemaphoreType.DMA,
            pltpu.SemaphoreType.DMA,
            pltpu.SemaphoreType.DMA,
        ],
    )(_sc_layer_body)


def _sc_layer(table, ea, src2d, dst2d):
    return _sc_layer_kernel()(table, ea, src2d, dst2d)


def _sc_layer_body(table_hbm, ea_hbm, src_hbm, dst_hbm, out_hbm,
                   idxs, idxd, gbuf, ebuf, acc, tbl, isem, gsem, esem, ssem):
    cid = lax.axis_index("c")
    sid = lax.axis_index("s")
    cb0 = cid * CPG + sid * CPT

    def zbody(r, _):
        zrow = jnp.zeros((16,), jnp.float32)
        gbuf[0, r, pl.ds(0, 16)] = zrow
        gbuf[0, r, pl.ds(16, 16)] = zrow
        gbuf[0, r, pl.ds(32, 16)] = zrow
        return 0

    lax.fori_loop(0, GROUP, zbody, 0)
    base = sid * SUB_ROWS
    for i in range(SUB_ROWS // GROUP):
        pltpu.sync_copy(gbuf.at[0], acc.at[pl.ds(base + i * GROUP, GROUP)])
    _rem = SUB_ROWS % GROUP
    if _rem:
        pltpu.sync_copy(gbuf.at[0].at[pl.ds(0, _rem)],
                        acc.at[pl.ds(base + SUB_ROWS - _rem, _rem)])
    pltpu.sync_copy(table_hbm.at[pl.ds(cid * N + sid * TBL_SUB, TBL_SUB)],
                    tbl.at[pl.ds(sid * TBL_SUB, TBL_SUB)])
    plsc.subcore_barrier()

    def fire_idxs(g, b):
        cb = cb0 + g * K
        pltpu.async_copy(src_hbm.at[pl.ds(cb, K)], idxs.at[b], isem)

    def fire_idxd(g, b):
        cb = cb0 + g * K
        pltpu.async_copy(dst_hbm.at[pl.ds(cb, K)], idxd.at[b], isem)

    def drain_idxs(g, b):
        cb = cb0 + g * K
        pltpu.make_async_copy(src_hbm.at[pl.ds(cb, K)], idxs.at[b], isem).wait()

    def drain_idxd(g, b):
        cb = cb0 + g * K
        pltpu.make_async_copy(dst_hbm.at[pl.ds(cb, K)], idxd.at[b], isem).wait()

    def fire_gathers(b):
        for j in range(K):
            pltpu.async_copy(tbl.at[idxs.at[b].at[j]],
                             gbuf.at[b].at[pl.ds(j * CH, CH)], gsem)

    def drain_gathers(b):
        for j in range(K):
            pltpu.make_async_copy(tbl.at[idxs.at[b].at[j]],
                                  gbuf.at[b].at[pl.ds(j * CH, CH)], gsem).wait()

    def fire_ea(g, b):
        rb = (cb0 + g * K) * CH
        pltpu.async_copy(ea_hbm.at[pl.ds(rb, GROUP), pl.ds(0, HP)],
                         ebuf.at[b], esem)

    def drain_ea(g, b):
        rb = (cb0 + g * K) * CH
        pltpu.make_async_copy(ea_hbm.at[pl.ds(rb, GROUP), pl.ds(0, HP)],
                              ebuf.at[b], esem).wait()

    def compute(b):
        gb = gbuf.at[b]
        eb = ebuf.at[b]

        def crow(r, _):
            for c in range(3):
                sl = pl.ds(c * 16, 16)
                gb[r, sl] = jnp.maximum(gb[r, sl] + eb[r, sl], 0.0)
            return 0

        lax.fori_loop(0, GROUP, crow, 0, unroll=4)

    def fire_scatters(b):
        for j in range(K):
            pltpu.async_copy(gbuf.at[b].at[pl.ds(j * CH, CH)],
                             acc.at[idxd.at[b].at[j]], ssem, add=True)

    def drain_scatters(b):
        for j in range(K):
            pltpu.make_async_copy(gbuf.at[b].at[pl.ds(j * CH, CH)],
                                  acc.at[idxd.at[b].at[j]], ssem).wait()

    def slot(g, b, first=False, fire_next=True):
        nb = 1 - b
        if fire_next:
            fire_idxs(g + 1, nb)
        drain_gathers(b)
        drain_ea(g, b)
        if not first:
            drain_scatters(nb)
        if fire_next:
            fire_idxd(g + 1, nb)
            drain_idxs(g + 1, nb)
            fire_gathers(nb)
            fire_ea(g + 1, nb)
            drain_idxd(g + 1, nb)
        compute(b)
        fire_scatters(b)

    fire_idxs(0, 0)
    fire_idxd(0, 0)
    drain_idxs(0, 0)
    fire_gathers(0)
    fire_ea(0, 0)
    drain_idxd(0, 0)

    slot(0, 0, first=True)
    slot(1, 1)

    def pair(p, _):
        g = 2 + 2 * p
        slot(g, 0)
        slot(g + 1, 1)
        return 0

    lax.fori_loop(0, (NGRP - 4) // 2, pair, 0)
    slot(NGRP - 2, 0)
    slot(NGRP - 1, 1, fire_next=False)

    drain_scatters(1)
    plsc.subcore_barrier()
    pltpu.sync_copy(acc.at[pl.ds(sid * SUB_ROWS, SUB_ROWS)],
                    out_hbm.at[cid, pl.ds(sid * SUB_ROWS, SUB_ROWS)])


def _enc_node_body(x_ref, w_ref, b_ref, o_ref):
    o_ref[...] = jnp.dot(x_ref[...], w_ref[...],
                         preferred_element_type=jnp.float32) + b_ref[...]


def _enc_node(x, w, b):
    bn = 2000
    return pl.pallas_call(
        _enc_node_body,
        grid=(NN // bn,),
        in_specs=[
            pl.BlockSpec((bn, DN), lambda i: (i, 0)),
            pl.BlockSpec((DN, HP), lambda i: (0, 0)),
            pl.BlockSpec((1, HP), lambda i: (0, 0)),
        ],
        out_specs=pl.BlockSpec((bn, HP), lambda i: (i, 0)),
        out_shape=jax.ShapeDtypeStruct((NN, HP), jnp.float32),
    )(x, w, b)


def _enc_edge_body(ea_ref, w_ref, b_ref, o_ref):
    o_ref[...] = jnp.dot(ea_ref[...], w_ref[...],
                         preferred_element_type=jnp.float32) + b_ref[...]


def _enc_edge(ea, w, b):
    be = 2560
    nvb = E // be
    npb = EPG_P // be

    def in_map(i):
        g = i // npb
        return (g * nvb + jnp.minimum(i % npb, nvb - 1), 0)

    return pl.pallas_call(
        _enc_edge_body,
        grid=(2 * npb,),
        in_specs=[
            pl.BlockSpec((be, DE), in_map),
            pl.BlockSpec((DE, 128), lambda i: (0, 0)),
            pl.BlockSpec((1, 128), lambda i: (0, 0)),
        ],
        out_specs=pl.BlockSpec((be, 128), lambda i: (i, 0)),
        out_shape=jax.ShapeDtypeStruct((ROWS_P, 128), jnp.float32),
    )(ea, w, b)


def _mlp_body(h_ref, p_ref, w1_ref, b1_ref, w2_ref, b2_ref, o_ref):
    t = h_ref[...] + p_ref[...]
    r = jnp.maximum(jnp.dot(t, w1_ref[...],
                            preferred_element_type=jnp.float32) + b1_ref[...], 0.0)
    o_ref[...] = jnp.dot(r, w2_ref[...],
                         preferred_element_type=jnp.float32) + b2_ref[...]


def _mlp(h, p, w1, b1, w2, b2):
    bn = 2000
    return pl.pallas_call(
        _mlp_body,
        grid=(NN // bn,),
        in_specs=[
            pl.BlockSpec((bn, HP), lambda i: (i, 0)),
            pl.BlockSpec((bn, HP), lambda i: (i, 0)),
            pl.BlockSpec((HP, 64), lambda i: (0, 0)),
            pl.BlockSpec((1, 64), lambda i: (0, 0)),
            pl.BlockSpec((64, HP), lambda i: (0, 0)),
            pl.BlockSpec((1, HP), lambda i: (0, 0)),
        ],
        out_specs=pl.BlockSpec((bn, HP), lambda i: (i, 0)),
        out_shape=jax.ShapeDtypeStruct((NN, HP), jnp.float32),
    )(h, p, w1, b1, w2, b2)


def _embed_body(b_ref, n1_ref, n2_ref, o_ref):
    i = pl.program_id(1)

    @pl.when(i == 0)
    def _():
        o_ref[...] = jnp.zeros_like(o_ref)

    iota = lax.broadcasted_iota(jnp.int32, (1, NG), 1).astype(jnp.float32)
    oh = (b_ref[...] == iota).astype(jnp.float32)
    dims = (((0,), (0,)), ((), ()))
    g1 = lax.dot_general(oh, n1_ref[...], dims,
                         preferred_element_type=jnp.float32)
    g2 = lax.dot_general(oh, n2_ref[...], dims,
                         preferred_element_type=jnp.float32)
    o_ref[0, :, 0:HP] += g1
    o_ref[0, :, HP:2 * HP] += g2


def _embed(batchf, nf1, nf2):
    bn = 2000
    nblk = N // bn
    return pl.pallas_call(
        _embed_body,
        grid=(2, nblk),
        in_specs=[
            pl.BlockSpec((bn, 1), lambda g, i: (g * nblk + i, 0)),
            pl.BlockSpec((bn, HP), lambda g, i: (g * nblk + i, 0)),
            pl.BlockSpec((bn, HP), lambda g, i: (g * nblk + i, 0)),
        ],
        out_specs=pl.BlockSpec((1, NG, 2 * HP), lambda g, i: (g, 0, 0)),
        out_shape=jax.ShapeDtypeStruct((2, NG, 2 * HP), jnp.float32),
    )(batchf, nf1, nf2)


def _mmd_body(emb_ref, o_ref):
    def term(xi, yi):
        X = emb_ref[xi]
        Y = emb_ref[yi]
        diff = X[:, None, :] - Y[None, :, :]
        s = jnp.sum(diff * diff, axis=-1)
        safe = jnp.where(s > 0, s, 1.0)
        d = jnp.where(s > 0, jnp.sqrt(safe), 0.0)
        return jnp.sum(jnp.exp(-0.5 * d))

    den = float(2 * HID) * float(2 * HID)
    total = (term(0, 0) + term(1, 1) - 2.0 * term(0, 1)) / den
    o_ref[...] = jnp.broadcast_to(total, (1, 1))


def _mmd(emb):
    return pl.pallas_call(
        _mmd_body,
        out_shape=jax.ShapeDtypeStruct((1, 1), jnp.float32),
    )(emb)


def kernel(x_real, edge_index_real, batch_real, edge_attr_real,
           x_gen, edge_index_gen, batch_gen, edge_attr_gen,
           W_node, b_node, W_edge, b_edge, W1, b1, W2, b2):
    pad = HP - HID
    wn = jnp.pad(W_node, ((0, 0), (0, pad)))
    bn = jnp.pad(b_node, (0, pad)).reshape(1, HP)
    we = jnp.pad(W_edge, ((0, 0), (0, 128 - HID)))
    be = jnp.pad(b_edge, (0, 128 - HID)).reshape(1, 128)
    w1 = jnp.pad(W1, ((0, pad), (0, 0)))
    b1r = b1.reshape(1, 64)
    w2 = jnp.pad(W2, ((0, 0), (0, pad)))
    b2r = jnp.pad(b2, (0, pad)).reshape(1, HP)

    x_all = jnp.concatenate([x_real, x_gen], axis=0)
    ea_all = jnp.concatenate([edge_attr_real, edge_attr_gen], axis=0)
    padn = EPG_P - E
    src2d = jnp.concatenate([
        jnp.pad(edge_index_real[0], (0, padn)),
        jnp.pad(edge_index_gen[0], (0, padn)),
    ]).reshape(NCHUNK, CH)
    dst2d = jnp.concatenate([
        jnp.pad(edge_index_real[1], (0, padn), constant_values=TRASH),
        jnp.pad(edge_index_gen[1], (0, padn), constant_values=TRASH),
    ]).reshape(NCHUNK, CH)
    batchf = jnp.concatenate([batch_real, batch_gen]).astype(jnp.float32)
    batchf = batchf.reshape(NN, 1)

    h = _enc_node(x_all, wn, bn)
    eaf = _enc_edge(ea_all, we, be)

    def gine_layer(t, _):
        parts = _sc_layer(t, eaf, src2d, dst2d)
        p = parts[:, :N].reshape(NN, HP)
        nf = _mlp(t, p, w1, b1r, w2, b2r)
        return nf, nf

    _, nfs = lax.scan(gine_layer, h, None, length=2)
    nf1, nf2 = nfs[0], nfs[1]

    emb = _embed(batchf, nf1, nf2)
    return _mmd(emb)[0, 0]
